# Initial kernel scaffold; baseline (speedup 1.0000x reference)
import jax
import jax.numpy as jnp
from jax.experimental import pallas as pl


def kernel(state, adj, W1, b1, W2, b2, Wf1, bf1, Wf2, bf2, Wf3, bf3, Wf4, bf4, Wpi, bpi, Wv, bv):
    raise NotImplementedError("write your pallas kernel here")


# speedup vs baseline: 1.8496x; 1.8496x over previous
import jax
import jax.numpy as jnp
from jax.experimental import pallas as pl

_N = 21
_F = 128


def _fused_fwd(state_ref, adj_ref, W1_ref, b1_ref, W2_ref, b2_ref,
               Wf1_ref, bf1_ref, Wf2_ref, bf2_ref, Wf3_ref, bf3_ref,
               Wf4_ref, bf4_ref, Wpi_ref, bpi_ref, Wv_ref, bv_ref,
               pi_ref, v_ref):
    x0 = state_ref[0]
    a = adj_ref[0]
    a = a + jnp.eye(_N, dtype=a.dtype)
    deg = jnp.sum(a, axis=1, keepdims=True)
    s = jnp.where(deg > 0, jax.lax.rsqrt(deg), 0.0)

    z = s * jnp.dot(x0, W1_ref[...], preferred_element_type=jnp.float32)
    x = s * jnp.dot(a, z, preferred_element_type=jnp.float32) + b1_ref[...]
    z = s * jnp.dot(x, W2_ref[...], preferred_element_type=jnp.float32)
    x = s * jnp.dot(a, z, preferred_element_type=jnp.float32) + b2_ref[...]

    h = bf1_ref[...]
    for i in range(_N):
        h = h + jnp.dot(x[i:i + 1, :], Wf1_ref[i * _N:(i + 1) * _N, :],
                        preferred_element_type=jnp.float32)
    h = jnp.maximum(h, 0.0)
    h = jnp.maximum(
        jnp.dot(h, Wf2_ref[...], preferred_element_type=jnp.float32)
        + bf2_ref[...], 0.0)
    vx = jnp.maximum(
        jnp.dot(h, Wf3_ref[...], preferred_element_type=jnp.float32)
        + bf3_ref[...], 0.0)
    vx = jnp.maximum(
        jnp.dot(vx, Wf4_ref[...], preferred_element_type=jnp.float32)
        + bf4_ref[...], 0.0)

    pi_ref[...] = (jnp.dot(h, Wpi_ref[...], preferred_element_type=jnp.float32)
                   + bpi_ref[...])
    v_ref[...] = (jnp.dot(vx, Wv_ref[...], preferred_element_type=jnp.float32)
                  + bv_ref[...])


def kernel(state, adj, W1, b1, W2, b2, Wf1, bf1, Wf2, bf2, Wf3, bf3,
           Wf4, bf4, Wpi, bpi, Wv, bv):
    full = lambda x: pl.BlockSpec(x.shape, lambda i: tuple(0 for _ in x.shape))
    in_specs = [
        pl.BlockSpec((1, _N, _F), lambda i: (0, 0, 0)),
        pl.BlockSpec((1, _N, _N), lambda i: (0, 0, 0)),
    ]
    b1r, b2r = b1.reshape(1, _N), b2.reshape(1, _N)
    bf1r, bf2r = bf1.reshape(1, -1), bf2.reshape(1, -1)
    bf3r, bf4r = bf3.reshape(1, -1), bf4.reshape(1, -1)
    bpir, bvr = bpi.reshape(1, -1), bv.reshape(1, 1)
    rest = [W1, b1r, W2, b2r, Wf1, bf1r, Wf2, bf2r, Wf3, bf3r,
            Wf4, bf4r, Wpi, bpir, Wv, bvr]
    in_specs += [full(x) for x in rest]

    pi, v = pl.pallas_call(
        _fused_fwd,
        out_shape=(jax.ShapeDtypeStruct((1, 64), jnp.float32),
                   jax.ShapeDtypeStruct((1, 1), jnp.float32)),
        grid=(1,),
        in_specs=in_specs,
        out_specs=(pl.BlockSpec((1, 64), lambda i: (0, 0)),
                   pl.BlockSpec((1, 1), lambda i: (0, 0))),
    )(state, adj, *rest)
    return pi.reshape(64), v.reshape(1)
